# baseline (speedup 1.0000x reference)
import functools

import jax
import jax.numpy as jnp
from jax import lax
from jax.experimental import pallas as pl
from jax.experimental.pallas import tpu as pltpu
from jax.experimental.pallas import tpu_sc as plsc

_B = 1024
_K = 5
_V = 100000
_D = 64
_H = 64
_A = 4
_NRW = 90000
_NH = 3
_GH = 256
_BC = _B + _B + _B * _K
_BLK = 512
_F32 = jnp.float32


def _gathers(stoich_tab, emb_tab, all_idx):
    info = plsc.get_sparse_core_info()
    nw = info.num_cores * info.num_subcores
    ps = _BC // nw

    mesh = plsc.VectorSubcoreMesh(core_axis_name="c", subcore_axis_name="s")

    @functools.partial(
        pl.kernel,
        mesh=mesh,
        out_type=[
            jax.ShapeDtypeStruct((_BC, 128), _F32),
            jax.ShapeDtypeStruct((_BC, 128), _F32),
        ],
        scratch_types=[
            pltpu.VMEM((ps,), jnp.int32),
            pltpu.VMEM((ps, 128), _F32),
            pltpu.VMEM((ps, 128), _F32),
            pltpu.SemaphoreType.DMA,
        ],
    )
    def k(st_hbm, em_hbm, aidx_hbm, o_st, o_em, iv, r1, r2, sem):
        wid = lax.axis_index("s") * info.num_cores + lax.axis_index("c")
        base = wid * ps
        pltpu.sync_copy(aidx_hbm.at[pl.ds(base, ps)], iv)
        pltpu.async_copy(st_hbm.at[iv], r1, sem).wait()
        pltpu.sync_copy(r1, o_st.at[pl.ds(base, ps)])
        pltpu.async_copy(em_hbm.at[iv], r2, sem).wait()
        pltpu.sync_copy(r2, o_em.at[pl.ds(base, ps)])

    return k(stoich_tab, emb_tab, all_idx)


def _leaky(x):
    return jnp.where(x >= 0, x, 0.01 * x)


def _net_body(rows_ref, et_ref, wemb_ref, bemb_ref, *args):
    wargs = list(args[:-1])
    out_ref = args[-1]
    layers = [wargs[7 * l: 7 * l + 7] for l in range(3)]
    w1c, b1c, wcg2t, bcg2, wcm2, bcm2 = wargs[21:27]
    wt, bt, wc, bc = wargs[27:31]

    types_f = rows_ref[:, 0:_A]
    wraw = rows_ref[:, _A:2 * _A]
    wn = wraw / (jnp.sum(wraw, axis=1, keepdims=True) + 1e-8)

    et2 = jnp.dot(et_ref[...], wemb_ref[...],
                  preferred_element_type=_F32)
    iota_e = lax.broadcasted_iota(jnp.int32, (_BLK, 100), 1).astype(_F32)
    elem = []
    for i in range(_A):
        oh = (iota_e == types_f[:, i:i + 1]).astype(_F32)
        elem.append(jnp.dot(oh, et2, preferred_element_type=_F32)
                    + bemb_ref[...])

    for (w1s, w1n, b1, wg2t, bg2, wm2, bm2) in layers:
        acc = [jnp.zeros((_BLK, _H), _F32) for _ in range(_A)]
        for h in range(_NH):
            sl = slice(h * 2 * _GH, (h + 1) * 2 * _GH)
            s_h = [jnp.dot(e, w1s[:, sl], preferred_element_type=_F32)
                   for e in elem]
            n_h = [jnp.dot(e, w1n[:, sl], preferred_element_type=_F32)
                   for e in elem]
            b1h = b1[:, sl]
            wm2h = wm2[h * _GH:(h + 1) * _GH, :]
            for i in range(_A):
                gs, ms, ws = [], [], []
                for j in range(_A):
                    if j == i:
                        continue
                    hid = _leaky(s_h[i] + n_h[j] + b1h)
                    g = (jnp.sum(hid[:, :_GH] * wg2t[h:h + 1, :],
                                 axis=1, keepdims=True)
                         + bg2[:, h:h + 1])
                    m = (jnp.dot(hid[:, _GH:], wm2h,
                                 preferred_element_type=_F32)
                         + bm2[h:h + 1, :])
                    gs.append(g); ms.append(m); ws.append(wn[:, j:j + 1])
                gmax = jnp.maximum(jnp.maximum(gs[0], gs[1]), gs[2])
                e0 = ws[0] * jnp.exp(gs[0] - gmax)
                e1 = ws[1] * jnp.exp(gs[1] - gmax)
                e2 = ws[2] * jnp.exp(gs[2] - gmax)
                den = e0 + e1 + e2 + 1e-10
                acc[i] = acc[i] + (e0 * ms[0] + e1 * ms[1] + e2 * ms[2]) / den
        elem = [elem[i] + acc[i] / float(_NH) for i in range(_A)]

    cacc = jnp.zeros((_BLK, _H), _F32)
    for h in range(_NH):
        sl = slice(h * 2 * _GH, (h + 1) * 2 * _GH)
        b1h = b1c[:, sl]
        wcm2h = wcm2[h * _GH:(h + 1) * _GH, :]
        gs, ms = [], []
        for i in range(_A):
            hid = _leaky(jnp.dot(elem[i], w1c[:, sl],
                                 preferred_element_type=_F32) + b1h)
            g = (jnp.sum(hid[:, :_GH] * wcg2t[h:h + 1, :],
                         axis=1, keepdims=True) + bcg2[:, h:h + 1])
            m = (jnp.dot(hid[:, _GH:], wcm2h,
                         preferred_element_type=_F32) + bcm2[h:h + 1, :])
            gs.append(g); ms.append(m)
        gmax = jnp.maximum(jnp.maximum(gs[0], gs[1]),
                           jnp.maximum(gs[2], gs[3]))
        es = [wn[:, i:i + 1] * jnp.exp(gs[i] - gmax) for i in range(_A)]
        den = es[0] + es[1] + es[2] + es[3] + 1e-10
        cacc = cacc + (es[0] * ms[0] + es[1] * ms[1]
                       + es[2] * ms[2] + es[3] * ms[3]) / den
    cry = cacc / float(_NH)

    out_t = jnp.dot(cry, wt[...], preferred_element_type=_F32) + bt[...]
    out_c = jnp.dot(cry, wc[...], preferred_element_type=_F32) + bc[...]
    f = (pl.program_id(0) * _BLK < _B).astype(_F32)
    out_ref[...] = f * out_t + (1.0 - f) * out_c


def _pack_edge_layer(layer):
    w1s = jnp.concatenate(
        [jnp.concatenate([h['gate'][0][:_H], h['msg'][0][:_H]], axis=1)
         for h in layer], axis=1)
    w1n = jnp.concatenate(
        [jnp.concatenate([h['gate'][0][_H:], h['msg'][0][_H:]], axis=1)
         for h in layer], axis=1)
    b1 = jnp.concatenate(
        [jnp.concatenate([h['gate'][1], h['msg'][1]]) for h in layer]
    ).reshape(1, 2 * _GH * _NH)
    wg2t = jnp.concatenate([h['gate'][2].T for h in layer], axis=0)
    bg2 = jnp.concatenate([h['gate'][3] for h in layer]).reshape(1, _NH)
    wm2 = jnp.concatenate([h['msg'][2] for h in layer], axis=0)
    bm2 = jnp.stack([h['msg'][3] for h in layer], axis=0)
    return [w1s, w1n, b1, wg2t, bg2, wm2, bm2]


def _pack_cry(heads):
    w1c = jnp.concatenate(
        [jnp.concatenate([h['gate'][0], h['msg'][0]], axis=1)
         for h in heads], axis=1)
    b1c = jnp.concatenate(
        [jnp.concatenate([h['gate'][1], h['msg'][1]]) for h in heads]
    ).reshape(1, 2 * _GH * _NH)
    wcg2t = jnp.concatenate([h['gate'][2].T for h in heads], axis=0)
    bcg2 = jnp.concatenate([h['gate'][3] for h in heads]).reshape(1, _NH)
    wcm2 = jnp.concatenate([h['msg'][2] for h in heads], axis=0)
    bcm2 = jnp.stack([h['msg'][3] for h in heads], axis=0)
    return [w1c, b1c, wcg2t, bcg2, wcm2, bcm2]


def _network(st_rows, params):
    wlist = []
    for layer in params['graph']:
        wlist += _pack_edge_layer(layer)
    wlist += _pack_cry(params['cry'])
    wlist += [params['W_tmeg'], params['b_tmeg'].reshape(1, _D),
              params['W_cmeg'], params['b_cmeg'].reshape(1, _D)]

    args = [st_rows, params['_elem_table'], params['W_embd'],
            params['b_embd'].reshape(1, _H)] + wlist
    grid = _BC // _BLK
    in_specs = [pl.BlockSpec((_BLK, 128), lambda i: (i, 0))]
    for a in args[1:]:
        in_specs.append(
            pl.BlockSpec(a.shape, lambda i: tuple(0 for _ in a.shape)))
    return pl.pallas_call(
        _net_body,
        grid=(grid,),
        in_specs=in_specs,
        out_specs=pl.BlockSpec((_BLK, _D), lambda i: (i, 0)),
        out_shape=jax.ShapeDtypeStruct((_BC, _D), _F32),
    )(*args)


def _softplus(x):
    return jnp.log(1.0 + jnp.exp(x))


def _score_body(euw_ref, eum_ref, evw_ref, evm_ref, enw_ref, enm_ref,
                pu_ref, pv_ref, nv_ref, out_ref):
    pu = pu_ref[...]
    pv = pv_ref[...]
    nv = nv_ref[...]
    u_opts = [(euw_ref[...], pu >= _NRW), (eum_ref[...], pu < _NRW)]
    v_opts = [(evw_ref[...], pv >= _NRW), (evm_ref[...], pv < _NRW)]
    n_opts = [(enw_ref[...], nv >= _NRW), (enm_ref[...], nv < _NRW)]
    s = 0
    for eu, um in u_opts:
        for ev, vm in v_opts:
            pos = jnp.sum(eu * ev, axis=1, keepdims=True)
            pos = jnp.clip(pos, -10.0, 10.0)
            posl = jnp.where(vm, 0.0, _softplus(-pos))
            for en, nm in n_opts:
                negsum = jnp.zeros_like(posl)
                for k in range(_K):
                    nk = jnp.sum(eu * en[:, _D * k:_D * (k + 1)],
                                 axis=1, keepdims=True)
                    nk = jnp.clip(nk, -10.0, 10.0)
                    negsum = negsum + jnp.where(nm[:, k:k + 1], 0.0,
                                                _softplus(nk))
                tot = jnp.where(um, 0.0, posl + negsum) * 0.5
                out_ref[s:s + 1, :] = jnp.broadcast_to(jnp.sum(tot), (1, 128))
                s += 1


def _scores(euw, eum, evw, evm, enw, enm, pu, pv, nv):
    args = [euw, eum, evw, evm, enw, enm, pu, pv, nv]
    return pl.pallas_call(
        _score_body,
        in_specs=[pl.BlockSpec(a.shape, lambda *_: tuple(0 for _ in a.shape))
                  for a in args],
        out_specs=pl.BlockSpec((8, 128), lambda *_: (0, 0)),
        out_shape=jax.ShapeDtypeStruct((8, 128), _F32),
    )(*args)


def kernel(pos_u, pos_v, neg_v, params, stoich_types, stoich_weights,
           elem_table):
    pu = pos_u.astype(jnp.int32)
    pv = pos_v.astype(jnp.int32)
    nv = neg_v.astype(jnp.int32)
    nvf = nv.reshape(-1)
    all_idx = jnp.concatenate([pu, pv, nvf])

    stoich_tab = jnp.concatenate(
        [stoich_types.astype(_F32), stoich_weights,
         jnp.zeros((_V, 120), _F32)], axis=1)
    emb_tab = jnp.concatenate([params['u_emb'], params['v_emb']],
                              axis=1)

    st_rows, emb_rows = _gathers(stoich_tab, emb_tab, all_idx)
    euw = emb_rows[:_B, :_D]
    evw = emb_rows[_B:2 * _B, _D:]
    enw = emb_rows[2 * _B:, _D:]

    p2 = dict(params)
    p2['_elem_table'] = elem_table
    emb_all = _network(st_rows, p2)
    eum = emb_all[:_B]
    evm = emb_all[_B:2 * _B]
    enm = emb_all[2 * _B:].reshape(_B, _K * _D)
    enw2 = enw.reshape(_B, _K * _D)

    sums = _scores(euw, eum, evw, evm, enw2, enm,
                   pu.reshape(_B, 1), pv.reshape(_B, 1), nv)
    sub_means = sums[:, 0] / float(_B)
    total = jnp.sum(sub_means)
    return total, sub_means


# speedup vs baseline: 1.1576x; 1.1576x over previous
import functools

import jax
import jax.numpy as jnp
from jax import lax
from jax.experimental import pallas as pl
from jax.experimental.pallas import tpu as pltpu
from jax.experimental.pallas import tpu_sc as plsc

_B = 1024
_K = 5
_V = 100000
_D = 64
_H = 64
_A = 4
_NRW = 90000
_NH = 3
_GH = 256
_BC = _B + _B + _B * _K
_BLK = 512
_F32 = jnp.float32


def _gathers(stoich_tab, emb_tab, all_idx):
    info = plsc.get_sparse_core_info()
    nw = info.num_cores * info.num_subcores
    ps = _BC // nw

    mesh = plsc.VectorSubcoreMesh(core_axis_name="c", subcore_axis_name="s")

    @functools.partial(
        pl.kernel,
        mesh=mesh,
        out_type=[
            jax.ShapeDtypeStruct((_BC, 128), _F32),
            jax.ShapeDtypeStruct((_BC, 128), _F32),
        ],
        scratch_types=[
            pltpu.VMEM((ps,), jnp.int32),
            pltpu.VMEM((ps, 128), _F32),
            pltpu.VMEM((ps, 128), _F32),
            pltpu.SemaphoreType.DMA,
        ],
    )
    def k(st_hbm, em_hbm, aidx_hbm, o_st, o_em, iv, r1, r2, sem):
        wid = lax.axis_index("s") * info.num_cores + lax.axis_index("c")
        base = wid * ps
        pltpu.sync_copy(aidx_hbm.at[pl.ds(base, ps)], iv)
        pltpu.async_copy(st_hbm.at[iv], r1, sem).wait()
        pltpu.sync_copy(r1, o_st.at[pl.ds(base, ps)])
        pltpu.async_copy(em_hbm.at[iv], r2, sem).wait()
        pltpu.sync_copy(r2, o_em.at[pl.ds(base, ps)])

    return k(stoich_tab, emb_tab, all_idx)


def _leaky(x):
    return jnp.where(x >= 0, x, 0.01 * x)


def _net_body(rows_ref, et_ref, wemb_ref, bemb_ref, *args):
    wargs = list(args[:-1])
    out_ref = args[-1]
    layers = [wargs[7 * l: 7 * l + 7] for l in range(3)]
    w1c, b1c, wcg2, bcg2, wcm2, bcm2 = wargs[21:27]
    wt, bt, wc, bc = wargs[27:31]

    types_f = rows_ref[:, 0:_A]
    wraw = rows_ref[:, _A:2 * _A]
    wn = wraw / (jnp.sum(wraw, axis=1, keepdims=True) + 1e-8)

    et2 = jnp.dot(et_ref[...], wemb_ref[...],
                  preferred_element_type=_F32)
    iota_e = lax.broadcasted_iota(jnp.int32, (_BLK, 100), 1).astype(_F32)
    elem = []
    for i in range(_A):
        oh = (iota_e == types_f[:, i:i + 1]).astype(_F32)
        elem.append(jnp.dot(oh, et2, preferred_element_type=_F32)
                    + bemb_ref[...])

    for (w1s, w1n, b1, wg2, bg2, wm2, bm2) in layers:
        acc = [jnp.zeros((_BLK, _H), _F32) for _ in range(_A)]
        for h in range(_NH):
            sl = slice(h * 2 * _GH, (h + 1) * 2 * _GH)
            b1h = b1[:, sl]
            s_h = [jnp.dot(e, w1s[:, sl], preferred_element_type=_F32) + b1h
                   for e in elem]
            n_h = [jnp.dot(e, w1n[:, sl], preferred_element_type=_F32)
                   for e in elem]
            wm2h = wm2[h * _GH:(h + 1) * _GH, :]
            for i in range(_A):
                gs, ms, ws = [], [], []
                for j in range(_A):
                    if j == i:
                        continue
                    pre = s_h[i] + n_h[j]
                    hid = jnp.maximum(pre, 0.01 * pre)
                    g = jnp.dot(hid[:, :_GH], wg2[:, h:h + 1],
                                preferred_element_type=_F32)
                    m = jnp.dot(hid[:, _GH:], wm2h,
                                preferred_element_type=_F32)
                    gs.append(g); ms.append(m); ws.append(wn[:, j:j + 1])
                gmax = jnp.maximum(jnp.maximum(gs[0], gs[1]), gs[2])
                e0 = ws[0] * jnp.exp(gs[0] - gmax)
                e1 = ws[1] * jnp.exp(gs[1] - gmax)
                e2 = ws[2] * jnp.exp(gs[2] - gmax)
                inv = 1.0 / (e0 + e1 + e2 + 1e-10)
                acc[i] = acc[i] + ((e0 * ms[0] + e1 * ms[1] + e2 * ms[2])
                                   * inv + bm2[h:h + 1, :])
        elem = [elem[i] + acc[i] / float(_NH) for i in range(_A)]

    cacc = jnp.zeros((_BLK, _H), _F32)
    for h in range(_NH):
        sl = slice(h * 2 * _GH, (h + 1) * 2 * _GH)
        b1h = b1c[:, sl]
        wcm2h = wcm2[h * _GH:(h + 1) * _GH, :]
        gs, ms = [], []
        for i in range(_A):
            pre = jnp.dot(elem[i], w1c[:, sl],
                          preferred_element_type=_F32) + b1h
            hid = jnp.maximum(pre, 0.01 * pre)
            g = jnp.dot(hid[:, :_GH], wcg2[:, h:h + 1],
                        preferred_element_type=_F32)
            m = jnp.dot(hid[:, _GH:], wcm2h,
                        preferred_element_type=_F32)
            gs.append(g); ms.append(m)
        gmax = jnp.maximum(jnp.maximum(gs[0], gs[1]),
                           jnp.maximum(gs[2], gs[3]))
        es = [wn[:, i:i + 1] * jnp.exp(gs[i] - gmax) for i in range(_A)]
        inv = 1.0 / (es[0] + es[1] + es[2] + es[3] + 1e-10)
        cacc = cacc + ((es[0] * ms[0] + es[1] * ms[1]
                        + es[2] * ms[2] + es[3] * ms[3]) * inv
                       + bcm2[h:h + 1, :])
    cry = cacc / float(_NH)

    out_t = jnp.dot(cry, wt[...], preferred_element_type=_F32) + bt[...]
    out_c = jnp.dot(cry, wc[...], preferred_element_type=_F32) + bc[...]
    f = (pl.program_id(0) * _BLK < _B).astype(_F32)
    out_ref[...] = f * out_t + (1.0 - f) * out_c


def _pack_edge_layer(layer):
    w1s = jnp.concatenate(
        [jnp.concatenate([h['gate'][0][:_H], h['msg'][0][:_H]], axis=1)
         for h in layer], axis=1)
    w1n = jnp.concatenate(
        [jnp.concatenate([h['gate'][0][_H:], h['msg'][0][_H:]], axis=1)
         for h in layer], axis=1)
    b1 = jnp.concatenate(
        [jnp.concatenate([h['gate'][1], h['msg'][1]]) for h in layer]
    ).reshape(1, 2 * _GH * _NH)
    wg2 = jnp.concatenate([h['gate'][2] for h in layer], axis=1)
    bg2 = jnp.concatenate([h['gate'][3] for h in layer]).reshape(1, _NH)
    wm2 = jnp.concatenate([h['msg'][2] for h in layer], axis=0)
    bm2 = jnp.stack([h['msg'][3] for h in layer], axis=0)
    return [w1s, w1n, b1, wg2, bg2, wm2, bm2]


def _pack_cry(heads):
    w1c = jnp.concatenate(
        [jnp.concatenate([h['gate'][0], h['msg'][0]], axis=1)
         for h in heads], axis=1)
    b1c = jnp.concatenate(
        [jnp.concatenate([h['gate'][1], h['msg'][1]]) for h in heads]
    ).reshape(1, 2 * _GH * _NH)
    wcg2 = jnp.concatenate([h['gate'][2] for h in heads], axis=1)
    bcg2 = jnp.concatenate([h['gate'][3] for h in heads]).reshape(1, _NH)
    wcm2 = jnp.concatenate([h['msg'][2] for h in heads], axis=0)
    bcm2 = jnp.stack([h['msg'][3] for h in heads], axis=0)
    return [w1c, b1c, wcg2, bcg2, wcm2, bcm2]


def _network(st_rows, params):
    wlist = []
    for layer in params['graph']:
        wlist += _pack_edge_layer(layer)
    wlist += _pack_cry(params['cry'])
    wlist += [params['W_tmeg'], params['b_tmeg'].reshape(1, _D),
              params['W_cmeg'], params['b_cmeg'].reshape(1, _D)]

    args = [st_rows, params['_elem_table'], params['W_embd'],
            params['b_embd'].reshape(1, _H)] + wlist
    grid = _BC // _BLK
    in_specs = [pl.BlockSpec((_BLK, 128), lambda i: (i, 0))]
    for a in args[1:]:
        in_specs.append(
            pl.BlockSpec(a.shape, lambda i: tuple(0 for _ in a.shape)))
    return pl.pallas_call(
        _net_body,
        grid=(grid,),
        in_specs=in_specs,
        out_specs=pl.BlockSpec((_BLK, _D), lambda i: (i, 0)),
        out_shape=jax.ShapeDtypeStruct((_BC, _D), _F32),
    )(*args)


def _softplus(x):
    return jnp.log(1.0 + jnp.exp(x))


def _score_body(euw_ref, eum_ref, evw_ref, evm_ref, enw_ref, enm_ref,
                pu_ref, pv_ref, nv_ref, out_ref):
    pu = pu_ref[...]
    pv = pv_ref[...]
    nv = nv_ref[...]
    u_opts = [(euw_ref[...], pu >= _NRW), (eum_ref[...], pu < _NRW)]
    v_opts = [(evw_ref[...], pv >= _NRW), (evm_ref[...], pv < _NRW)]
    n_opts = [(enw_ref[...], nv >= _NRW), (enm_ref[...], nv < _NRW)]
    s = 0
    for eu, um in u_opts:
        for ev, vm in v_opts:
            pos = jnp.sum(eu * ev, axis=1, keepdims=True)
            pos = jnp.clip(pos, -10.0, 10.0)
            posl = jnp.where(vm, 0.0, _softplus(-pos))
            for en, nm in n_opts:
                negsum = jnp.zeros_like(posl)
                for k in range(_K):
                    nk = jnp.sum(eu * en[:, _D * k:_D * (k + 1)],
                                 axis=1, keepdims=True)
                    nk = jnp.clip(nk, -10.0, 10.0)
                    negsum = negsum + jnp.where(nm[:, k:k + 1], 0.0,
                                                _softplus(nk))
                tot = jnp.where(um, 0.0, posl + negsum) * 0.5
                out_ref[s:s + 1, :] = jnp.broadcast_to(jnp.sum(tot), (1, 128))
                s += 1


def _scores(euw, eum, evw, evm, enw, enm, pu, pv, nv):
    args = [euw, eum, evw, evm, enw, enm, pu, pv, nv]
    return pl.pallas_call(
        _score_body,
        in_specs=[pl.BlockSpec(a.shape, lambda *_: tuple(0 for _ in a.shape))
                  for a in args],
        out_specs=pl.BlockSpec((8, 128), lambda *_: (0, 0)),
        out_shape=jax.ShapeDtypeStruct((8, 128), _F32),
    )(*args)


def kernel(pos_u, pos_v, neg_v, params, stoich_types, stoich_weights,
           elem_table):
    pu = pos_u.astype(jnp.int32)
    pv = pos_v.astype(jnp.int32)
    nv = neg_v.astype(jnp.int32)
    nvf = nv.reshape(-1)
    all_idx = jnp.concatenate([pu, pv, nvf])

    stoich_tab = jnp.concatenate(
        [stoich_types.astype(_F32), stoich_weights,
         jnp.zeros((_V, 120), _F32)], axis=1)
    emb_tab = jnp.concatenate([params['u_emb'], params['v_emb']],
                              axis=1)

    st_rows, emb_rows = _gathers(stoich_tab, emb_tab, all_idx)
    euw = emb_rows[:_B, :_D]
    evw = emb_rows[_B:2 * _B, _D:]
    enw = emb_rows[2 * _B:, _D:]

    p2 = dict(params)
    p2['_elem_table'] = elem_table
    emb_all = _network(st_rows, p2)
    eum = emb_all[:_B]
    evm = emb_all[_B:2 * _B]
    enm = emb_all[2 * _B:].reshape(_B, _K * _D)
    enw2 = enw.reshape(_B, _K * _D)

    sums = _scores(euw, eum, evw, evm, enw2, enm,
                   pu.reshape(_B, 1), pv.reshape(_B, 1), nv)
    sub_means = sums[:, 0] / float(_B)
    total = jnp.sum(sub_means)
    return total, sub_means
